# baseline (speedup 1.0000x reference)
import jax
import jax.numpy as jnp
from jax import lax
from jax.experimental import pallas as pl
from jax.experimental.pallas import tpu as pltpu
from jax.experimental.pallas import tpu_sc as plsc

V = 1000000
D = 32
B = 16384
NC = 2
NS = 16
L = 16
NW = NC * NS
BPW = B // NW
CH = 128
NCHUNK = BPW // CH
NGRP = CH // L
W = 4 * D


def _rsqrt(x):
    i = plsc.bitcast(x, jnp.int32)
    i = jnp.int32(0x5F3759DF) - (i >> 1)
    y = plsc.bitcast(i, jnp.float32)
    for _ in range(3):
        y = y * (1.5 - 0.5 * x * y * y)
    return y


def _sc_body(src_hbm, anc_hbm, rb_hbm, emb1_hbm, emb2_hbm, out_hbm,
             sidx_v, aidx_v, srow_v, arow_v, rb_v,
             g1_v, g2_v, h1_v, h2_v, out_v, sem):
    wid = lax.axis_index("s") * NC + lax.axis_index("c")
    base = wid * BPW

    pltpu.sync_copy(src_hbm.at[pl.ds(base, BPW)], sidx_v)
    pltpu.sync_copy(anc_hbm.at[pl.ds(base, BPW)], aidx_v)
    pltpu.sync_copy(rb_hbm, rb_v)

    def rowprep(i, carry):
        s = sidx_v[pl.ds(i * L, L)]
        a = aidx_v[pl.ds(i * L, L)]
        srow_v[pl.ds(i * L, L)] = s >> 2
        arow_v[pl.ds(i * L, L)] = a >> 2
        return carry

    lax.fori_loop(0, BPW // L, rowprep, 0)

    iota = lax.iota(jnp.int32, L)
    zero = jnp.zeros((L,), jnp.float32)

    def chunk(c, carry):
        coff = c * CH
        c1 = pltpu.async_copy(emb1_hbm.at[srow_v.at[pl.ds(coff, CH)]], g1_v, sem)
        c2 = pltpu.async_copy(emb2_hbm.at[srow_v.at[pl.ds(coff, CH)]], g2_v, sem)
        c3 = pltpu.async_copy(emb1_hbm.at[arow_v.at[pl.ds(coff, CH)]], h1_v, sem)
        c4 = pltpu.async_copy(emb2_hbm.at[arow_v.at[pl.ds(coff, CH)]], h2_v, sem)
        c1.wait()
        c2.wait()
        c3.wait()
        c4.wait()

        def group(g, carry2):
            rows = g * L + iota
            s = sidx_v[pl.ds(coff + g * L, L)]
            a = aidx_v[pl.ds(coff + g * L, L)]
            colS = (s & 3) * D
            colA = (a & 3) * D
            d1 = d2 = d12 = n1 = n2 = n12 = m1 = m2 = zero
            for d in range(D):
                cS = colS + d
                cA = colA + d
                g1d = plsc.load_gather(g1_v, [rows, cS])
                g2d = plsc.load_gather(g2_v, [rows, cS])
                h1d = plsc.load_gather(h1_v, [rows, cA])
                h2d = plsc.load_gather(h2_v, [rows, cA])
                r1d = rb_v[pl.ds(d * L, L)]
                r2d = rb_v[pl.ds((D + d) * L, L)]
                r12d = rb_v[pl.ds((2 * D + d) * L, L)]
                x1 = g1d * r1d
                x2 = g2d * r2d
                x12 = g1d * r12d * g2d * h1d
                d1 = d1 + x1 * h1d
                n1 = n1 + x1 * x1
                m1 = m1 + h1d * h1d
                d2 = d2 + x2 * h2d
                n2 = n2 + x2 * x2
                m2 = m2 + h2d * h2d
                d12 = d12 + x12 * h2d
                n12 = n12 + x12 * x12
            res = (d1 * _rsqrt(jnp.maximum(n1 * m1, 1e-24))
                   + d2 * _rsqrt(jnp.maximum(n2 * m2, 1e-24))
                   - d12 * _rsqrt(jnp.maximum(n12 * m2, 1e-24)))
            out_v[pl.ds(coff + g * L, L)] = res
            return carry2

        lax.fori_loop(0, NGRP, group, 0)
        return carry

    lax.fori_loop(0, NCHUNK, chunk, 0)
    pltpu.sync_copy(out_v, out_hbm.at[pl.ds(base, BPW)])


_sc_call = pl.kernel(
    _sc_body,
    out_type=jax.ShapeDtypeStruct((B,), jnp.float32),
    mesh=plsc.VectorSubcoreMesh(core_axis_name="c", subcore_axis_name="s",
                                num_cores=NC, num_subcores=NS),
    compiler_params=pltpu.CompilerParams(needs_layout_passes=False),
    scratch_types=[
        pltpu.VMEM((BPW,), jnp.int32),
        pltpu.VMEM((BPW,), jnp.int32),
        pltpu.VMEM((BPW,), jnp.int32),
        pltpu.VMEM((BPW,), jnp.int32),
        pltpu.VMEM((3 * D * L,), jnp.float32),
        pltpu.VMEM((CH, W), jnp.float32),
        pltpu.VMEM((CH, W), jnp.float32),
        pltpu.VMEM((CH, W), jnp.float32),
        pltpu.VMEM((CH, W), jnp.float32),
        pltpu.VMEM((BPW,), jnp.float32),
        pltpu.SemaphoreType.DMA,
    ],
)


def kernel(source_nodes, anchor_nodes, rel_id, emb1, emb2, rvecs1, rvecs2):
    src = source_nodes.astype(jnp.int32)
    anc = anchor_nodes.astype(jnp.int32)
    r1 = rvecs1[rel_id]
    r2 = rvecs2[rel_id]
    rb = jnp.concatenate([
        jnp.broadcast_to(r1[:, None], (D, L)),
        jnp.broadcast_to(r2[:, None], (D, L)),
        jnp.broadcast_to((r1 * r2)[:, None], (D, L)),
    ], axis=0).reshape(3 * D * L)
    e1 = emb1.reshape(V // 4, W)
    e2 = emb2.reshape(V // 4, W)
    return _sc_call(src, anc, rb, e1, e2)


# speedup vs baseline: 1.9680x; 1.9680x over previous
import jax
import jax.numpy as jnp
from jax import lax
from jax.experimental import pallas as pl
from jax.experimental.pallas import tpu as pltpu
from jax.experimental.pallas import tpu_sc as plsc

V = 1000000
D = 32
B = 16384
NC = 2
NS = 16
L = 16
NW = NC * NS
BPW = B // NW
NGRP = BPW // L
VG = V // 8


def _rsqrt(x):
    i = plsc.bitcast(x, jnp.int32)
    i = jnp.int32(0x5F3759DF) - (i >> 1)
    y = plsc.bitcast(i, jnp.float32)
    for _ in range(3):
        y = y * (1.5 - 0.5 * x * y * y)
    return y


def _sc_body(src_hbm, anc_hbm, rb_hbm, emb1_hbm, emb2_hbm, out_hbm,
             sidx_v, aidx_v, rb_v, g1_v, g2_v, h1_v, h2_v, out_v, sem):
    wid = lax.axis_index("s") * NC + lax.axis_index("c")
    base = wid * BPW

    pltpu.sync_copy(src_hbm.at[pl.ds(base, BPW)], sidx_v)
    pltpu.sync_copy(anc_hbm.at[pl.ds(base, BPW)], aidx_v)
    pltpu.sync_copy(rb_hbm, rb_v)

    iota = lax.iota(jnp.int32, L)
    zero = jnp.zeros((L,), jnp.float32)

    def group(g, carry):
        sv = sidx_v[pl.ds(g * L, L)]
        av = aidx_v[pl.ds(g * L, L)]
        sg = sv >> 3
        ag = av >> 3
        subS = sv & 7
        subA = av & 7

        copies = []
        for l in range(L):
            s = sg[l]
            a = ag[l]
            copies.append(pltpu.async_copy(emb1_hbm.at[s], g1_v.at[l], sem))
            copies.append(pltpu.async_copy(emb2_hbm.at[s], g2_v.at[l], sem))
            copies.append(pltpu.async_copy(emb1_hbm.at[a], h1_v.at[l], sem))
            copies.append(pltpu.async_copy(emb2_hbm.at[a], h2_v.at[l], sem))
        for c in copies:
            c.wait()

        d1 = d2 = d12 = n1 = n2 = n12 = m1 = m2 = zero
        for d in range(D):
            col = jnp.full((L,), d, jnp.int32)
            g1d = plsc.load_gather(g1_v, [iota, subS, col])
            g2d = plsc.load_gather(g2_v, [iota, subS, col])
            h1d = plsc.load_gather(h1_v, [iota, subA, col])
            h2d = plsc.load_gather(h2_v, [iota, subA, col])
            r1d = rb_v[pl.ds(d * L, L)]
            r2d = rb_v[pl.ds((D + d) * L, L)]
            r12d = rb_v[pl.ds((2 * D + d) * L, L)]
            x1 = g1d * r1d
            x2 = g2d * r2d
            x12 = g1d * r12d * g2d * h1d
            d1 = d1 + x1 * h1d
            n1 = n1 + x1 * x1
            m1 = m1 + h1d * h1d
            d2 = d2 + x2 * h2d
            n2 = n2 + x2 * x2
            m2 = m2 + h2d * h2d
            d12 = d12 + x12 * h2d
            n12 = n12 + x12 * x12
        res = (d1 * _rsqrt(jnp.maximum(n1 * m1, 1e-24))
               + d2 * _rsqrt(jnp.maximum(n2 * m2, 1e-24))
               - d12 * _rsqrt(jnp.maximum(n12 * m2, 1e-24)))
        out_v[pl.ds(g * L, L)] = res
        return carry

    lax.fori_loop(0, NGRP, group, 0)
    pltpu.sync_copy(out_v, out_hbm.at[pl.ds(base, BPW)])


_sc_call = pl.kernel(
    _sc_body,
    out_type=jax.ShapeDtypeStruct((B,), jnp.float32),
    mesh=plsc.VectorSubcoreMesh(core_axis_name="c", subcore_axis_name="s",
                                num_cores=NC, num_subcores=NS),
    compiler_params=pltpu.CompilerParams(needs_layout_passes=False),
    scratch_types=[
        pltpu.VMEM((BPW,), jnp.int32),
        pltpu.VMEM((BPW,), jnp.int32),
        pltpu.VMEM((3 * D * L,), jnp.float32),
        pltpu.VMEM((L, 8, D), jnp.float32),
        pltpu.VMEM((L, 8, D), jnp.float32),
        pltpu.VMEM((L, 8, D), jnp.float32),
        pltpu.VMEM((L, 8, D), jnp.float32),
        pltpu.VMEM((BPW,), jnp.float32),
        pltpu.SemaphoreType.DMA,
    ],
)


def kernel(source_nodes, anchor_nodes, rel_id, emb1, emb2, rvecs1, rvecs2):
    src = source_nodes.astype(jnp.int32)
    anc = anchor_nodes.astype(jnp.int32)
    r1 = rvecs1[rel_id]
    r2 = rvecs2[rel_id]
    rb = jnp.concatenate([
        jnp.broadcast_to(r1[:, None], (D, L)),
        jnp.broadcast_to(r2[:, None], (D, L)),
        jnp.broadcast_to((r1 * r2)[:, None], (D, L)),
    ], axis=0).reshape(3 * D * L)
    e1 = emb1.reshape(VG, 8, D)
    e2 = emb2.reshape(VG, 8, D)
    return _sc_call(src, anc, rb, e1, e2)


# speedup vs baseline: 2.0292x; 1.0311x over previous
import jax
import jax.numpy as jnp
from jax import lax
from jax.experimental import pallas as pl
from jax.experimental.pallas import tpu as pltpu
from jax.experimental.pallas import tpu_sc as plsc

V = 1000000
D = 32
B = 16384
NC = 2
NS = 16
L = 16
NW = NC * NS
BPW = B // NW
NGRP = BPW // L
VG = V // 8


def _rsqrt(x):
    i = plsc.bitcast(x, jnp.int32)
    i = jnp.int32(0x5F3759DF) - (i >> 1)
    y = plsc.bitcast(i, jnp.float32)
    for _ in range(3):
        y = y * (1.5 - 0.5 * x * y * y)
    return y


def _sc_body(src_hbm, anc_hbm, rb_hbm, emb1_hbm, emb2_hbm, out_hbm,
             sidx_v, aidx_v, rb_v,
             g1a_v, g2a_v, h1a_v, sem_a,
             g1b_v, g2b_v, h1b_v, sem_b,
             h2_v, sem_h, out_v):
    wid = lax.axis_index("s") * NC + lax.axis_index("c")
    base = wid * BPW

    pltpu.sync_copy(src_hbm.at[pl.ds(base, BPW)], sidx_v)
    pltpu.sync_copy(anc_hbm.at[pl.ds(base, BPW)], aidx_v)
    pltpu.sync_copy(rb_hbm, rb_v)

    iota = lax.iota(jnp.int32, L)
    zero = jnp.zeros((L,), jnp.float32)
    bufs = ((g1a_v, g2a_v, h1a_v, sem_a), (g1b_v, g2b_v, h1b_v, sem_b))

    def fire3(g, bs):
        g1_v, g2_v, h1_v, sem = bs
        sg = sidx_v[pl.ds(g * L, L)] >> 3
        ag = aidx_v[pl.ds(g * L, L)] >> 3
        for l in range(L):
            s = sg[l]
            a = ag[l]
            pltpu.async_copy(emb1_hbm.at[s], g1_v.at[l], sem)
            pltpu.async_copy(emb2_hbm.at[s], g2_v.at[l], sem)
            pltpu.async_copy(emb1_hbm.at[a], h1_v.at[l], sem)

    def fire_h2(g):
        ag = aidx_v[pl.ds(g * L, L)] >> 3
        for l in range(L):
            pltpu.async_copy(emb2_hbm.at[ag[l]], h2_v.at[l], sem_h)

    def consume(g, bs):
        g1_v, g2_v, h1_v, sem = bs
        dummy = emb1_hbm.at[pl.ds(0, L)]
        for buf in (g1_v, g2_v, h1_v):
            pltpu.make_async_copy(dummy, buf, sem).wait()
        pltpu.make_async_copy(dummy, h2_v, sem_h).wait()
        sv = sidx_v[pl.ds(g * L, L)]
        av = aidx_v[pl.ds(g * L, L)]
        subS = sv & 7
        subA = av & 7
        d1 = d2 = d12 = n1 = n2 = n12 = m1 = m2 = zero
        for d in range(D):
            col = jnp.full((L,), d, jnp.int32)
            g1d = plsc.load_gather(g1_v, [iota, subS, col])
            g2d = plsc.load_gather(g2_v, [iota, subS, col])
            h1d = plsc.load_gather(h1_v, [iota, subA, col])
            h2d = plsc.load_gather(h2_v, [iota, subA, col])
            r1d = rb_v[pl.ds(d * L, L)]
            r2d = rb_v[pl.ds((D + d) * L, L)]
            r12d = rb_v[pl.ds((2 * D + d) * L, L)]
            x1 = g1d * r1d
            x2 = g2d * r2d
            x12 = g1d * r12d * g2d * h1d
            d1 = d1 + x1 * h1d
            n1 = n1 + x1 * x1
            m1 = m1 + h1d * h1d
            d2 = d2 + x2 * h2d
            n2 = n2 + x2 * x2
            m2 = m2 + h2d * h2d
            d12 = d12 + x12 * h2d
            n12 = n12 + x12 * x12
        res = (d1 * _rsqrt(jnp.maximum(n1 * m1, 1e-24))
               + d2 * _rsqrt(jnp.maximum(n2 * m2, 1e-24))
               - d12 * _rsqrt(jnp.maximum(n12 * m2, 1e-24)))
        out_v[pl.ds(g * L, L)] = res

    fire3(0, bufs[0])

    def pair(g2, carry):
        g = g2 * 2
        fire_h2(g)
        fire3(g + 1, bufs[1])
        consume(g, bufs[0])
        fire_h2(g + 1)

        @pl.when(g2 < NGRP // 2 - 1)
        def _():
            fire3(g + 2, bufs[0])

        consume(g + 1, bufs[1])
        return carry

    lax.fori_loop(0, NGRP // 2, pair, 0)
    pltpu.sync_copy(out_v, out_hbm.at[pl.ds(base, BPW)])


_sc_call = pl.kernel(
    _sc_body,
    out_type=jax.ShapeDtypeStruct((B,), jnp.float32),
    mesh=plsc.VectorSubcoreMesh(core_axis_name="c", subcore_axis_name="s",
                                num_cores=NC, num_subcores=NS),
    compiler_params=pltpu.CompilerParams(needs_layout_passes=False),
    scratch_types=[
        pltpu.VMEM((BPW,), jnp.int32),
        pltpu.VMEM((BPW,), jnp.int32),
        pltpu.VMEM((3 * D * L,), jnp.float32),
        pltpu.VMEM((L, 8, D), jnp.float32),
        pltpu.VMEM((L, 8, D), jnp.float32),
        pltpu.VMEM((L, 8, D), jnp.float32),
        pltpu.SemaphoreType.DMA,
        pltpu.VMEM((L, 8, D), jnp.float32),
        pltpu.VMEM((L, 8, D), jnp.float32),
        pltpu.VMEM((L, 8, D), jnp.float32),
        pltpu.SemaphoreType.DMA,
        pltpu.VMEM((L, 8, D), jnp.float32),
        pltpu.SemaphoreType.DMA,
        pltpu.VMEM((BPW,), jnp.float32),
    ],
)


def kernel(source_nodes, anchor_nodes, rel_id, emb1, emb2, rvecs1, rvecs2):
    src = source_nodes.astype(jnp.int32)
    anc = anchor_nodes.astype(jnp.int32)
    r1 = rvecs1[rel_id]
    r2 = rvecs2[rel_id]
    rb = jnp.concatenate([
        jnp.broadcast_to(r1[:, None], (D, L)),
        jnp.broadcast_to(r2[:, None], (D, L)),
        jnp.broadcast_to((r1 * r2)[:, None], (D, L)),
    ], axis=0).reshape(3 * D * L)
    e1 = emb1.reshape(VG, 8, D)
    e2 = emb2.reshape(VG, 8, D)
    return _sc_call(src, anc, rb, e1, e2)
